# baseline (speedup 1.0000x reference)
import jax
import jax.numpy as jnp
from jax.experimental import pallas as pl
from jax.experimental.pallas import tpu as pltpu


def _contract_last(x, w):
    return jax.lax.dot_general(
        x, w, dimension_numbers=(((1,), (1,)), ((), ())),
        preferred_element_type=jnp.float32)


def _fused_kernel(seg_ref, enc_ref, wih_ref, whh_ref, bih_ref, bhh_ref,
                  gc_ref, out_ref, gi_ref, h_ref):
    B, G, H3 = gi_ref.shape
    H = H3 // 3
    i = pl.program_id(0)

    @pl.when(i < B)
    def _phase_a():
        seg = seg_ref[0].astype(jnp.float32)
        denom = jnp.sum(seg, axis=1, keepdims=True) + 1.0
        alpha = seg / denom
        inps = jnp.dot(alpha, enc_ref[0],
                       preferred_element_type=jnp.float32)
        gi = _contract_last(inps.astype(jnp.bfloat16), wih_ref[...])
        gi_ref[i] = gi + bih_ref[...]

    @pl.when(i == B)
    def _init_h():
        h_ref[...] = jnp.zeros_like(h_ref)

    @pl.when(i >= B)
    def _phase_b():
        t = i - B
        h = h_ref[...]
        gi = gi_ref[:, t, :]
        gh = _contract_last(h.astype(jnp.bfloat16),
                            whh_ref[...]) + bhh_ref[...]
        r = jax.nn.sigmoid(gi[:, :H] + gh[:, :H])
        z = jax.nn.sigmoid(gi[:, H:2 * H] + gh[:, H:2 * H])
        n = jnp.tanh(gi[:, 2 * H:] + r * gh[:, 2 * H:])
        h_new = (1.0 - z) * n + z * h
        mask = t < gc_ref[...]
        out_ref[0] = jnp.where(mask, h_new, 0.0)
        h_ref[...] = jnp.where(mask, h_new, h)


def kernel(enc_states, segments, group_count, W_ih, W_hh, b_ih, b_hh):
    B, S, D = enc_states.shape
    G = segments.shape[1]
    H = W_hh.shape[1]

    wih = W_ih.astype(jnp.bfloat16)
    whh = W_hh.astype(jnp.bfloat16)
    bih = b_ih.reshape(1, 3 * H)
    bhh = b_hh.reshape(1, 3 * H)
    gc = group_count.reshape(B, 1)

    last_a = B - 1
    out = pl.pallas_call(
        _fused_kernel,
        grid=(B + G,),
        in_specs=[
            pl.BlockSpec((1, G, S), lambda i: (jnp.minimum(i, last_a), 0, 0)),
            pl.BlockSpec((1, S, D), lambda i: (jnp.minimum(i, last_a), 0, 0)),
            pl.BlockSpec((3 * H, D), lambda i: (0, 0)),
            pl.BlockSpec((3 * H, H), lambda i: (0, 0)),
            pl.BlockSpec((1, 3 * H), lambda i: (0, 0)),
            pl.BlockSpec((1, 3 * H), lambda i: (0, 0)),
            pl.BlockSpec((B, 1), lambda i: (0, 0)),
        ],
        out_specs=pl.BlockSpec((1, B, H),
                               lambda i: (jnp.maximum(i - B, 0), 0, 0)),
        out_shape=jax.ShapeDtypeStruct((G, B, H), jnp.float32),
        scratch_shapes=[
            pltpu.VMEM((B, G, 3 * H), jnp.float32),
            pltpu.VMEM((B, H), jnp.float32),
        ],
        compiler_params=pltpu.CompilerParams(
            dimension_semantics=("arbitrary",)),
    )(segments, enc_states, wih, whh, bih, bhh, gc)
    return jnp.swapaxes(out, 0, 1)


# speedup vs baseline: 1.0588x; 1.0588x over previous
import jax
import jax.numpy as jnp
from jax.experimental import pallas as pl
from jax.experimental.pallas import tpu as pltpu


def _contract_last(x, w):
    return jax.lax.dot_general(
        x, w, dimension_numbers=(((1,), (1,)), ((), ())),
        preferred_element_type=jnp.float32)


def _fused_kernel(seg_ref, enc_ref, wih_ref, whh_ref, bih_ref, bhh_ref,
                  gc_ref, out_ref, gi_ref, h_ref):
    B, G, H3 = gi_ref.shape
    H = H3 // 3
    i = pl.program_id(0)

    @pl.when(i < B)
    def _phase_a():
        seg = seg_ref[0]
        segb = seg.astype(jnp.bfloat16)
        pooled = jnp.dot(segb, enc_ref[0].astype(jnp.bfloat16),
                         preferred_element_type=jnp.float32)
        denom = jnp.sum(seg, axis=1, keepdims=True).astype(jnp.float32)
        inps = pooled * (1.0 / (denom + 1.0))
        gi = _contract_last(inps.astype(jnp.bfloat16), wih_ref[...])
        gi_ref[i] = gi + bih_ref[...]

    @pl.when(i == B)
    def _init_h():
        h_ref[...] = jnp.zeros_like(h_ref)

    @pl.when(i >= B)
    def _phase_b():
        t = i - B
        h = h_ref[...]
        gi = gi_ref[:, t, :]
        gh = _contract_last(h.astype(jnp.bfloat16),
                            whh_ref[...]) + bhh_ref[...]
        r = jax.nn.sigmoid(gi[:, :H] + gh[:, :H])
        z = jax.nn.sigmoid(gi[:, H:2 * H] + gh[:, H:2 * H])
        n = jnp.tanh(gi[:, 2 * H:] + r * gh[:, 2 * H:])
        h_new = (1.0 - z) * n + z * h
        mask = t < gc_ref[...]
        out_ref[:, t, :] = jnp.where(mask, h_new, 0.0)
        h_ref[...] = jnp.where(mask, h_new, h)


def kernel(enc_states, segments, group_count, W_ih, W_hh, b_ih, b_hh):
    B, S, D = enc_states.shape
    G = segments.shape[1]
    H = W_hh.shape[1]

    wih = W_ih.astype(jnp.bfloat16)
    whh = W_hh.astype(jnp.bfloat16)
    bih = b_ih.reshape(1, 3 * H)
    bhh = b_hh.reshape(1, 3 * H)
    gc = group_count.reshape(B, 1)

    last_a = B - 1
    out = pl.pallas_call(
        _fused_kernel,
        grid=(B + G,),
        in_specs=[
            pl.BlockSpec((1, G, S), lambda i: (jnp.minimum(i, last_a), 0, 0)),
            pl.BlockSpec((1, S, D), lambda i: (jnp.minimum(i, last_a), 0, 0)),
            pl.BlockSpec((3 * H, D), lambda i: (0, 0)),
            pl.BlockSpec((3 * H, H), lambda i: (0, 0)),
            pl.BlockSpec((1, 3 * H), lambda i: (0, 0)),
            pl.BlockSpec((1, 3 * H), lambda i: (0, 0)),
            pl.BlockSpec((B, 1), lambda i: (0, 0)),
        ],
        out_specs=pl.BlockSpec((B, G, H), lambda i: (0, 0, 0)),
        out_shape=jax.ShapeDtypeStruct((B, G, H), jnp.float32),
        scratch_shapes=[
            pltpu.VMEM((B, G, 3 * H), jnp.float32),
            pltpu.VMEM((B, H), jnp.float32),
        ],
        compiler_params=pltpu.CompilerParams(
            dimension_semantics=("arbitrary",)),
    )(segments, enc_states, wih, whh, bih, bhh, gc)
    return out


# speedup vs baseline: 1.3096x; 1.2369x over previous
import jax
import jax.numpy as jnp
from jax.experimental import pallas as pl
from jax.experimental.pallas import tpu as pltpu

_CB = 4
_CT = 2


def _contract_last(x, w):
    return jax.lax.dot_general(
        x, w, dimension_numbers=(((1,), (1,)), ((), ())),
        preferred_element_type=jnp.float32)


def _fused_kernel(seg_ref, enc_ref, wih_ref, whh_ref, bih_ref, bhh_ref,
                  gc_ref, out_ref, gi_ref, h_ref):
    B, G, H3 = gi_ref.shape
    H = H3 // 3
    na = B // _CB
    i = pl.program_id(0)

    @pl.when(i < na)
    def _phase_a():
        inps = []
        for j in range(_CB):
            seg = seg_ref[j]
            segb = seg.astype(jnp.bfloat16)
            pooled = jnp.dot(segb, enc_ref[j].astype(jnp.bfloat16),
                             preferred_element_type=jnp.float32)
            denom = jnp.sum(seg, axis=1, keepdims=True).astype(jnp.float32)
            inps.append(pooled * (1.0 / (denom + 1.0)))
        chunk = jnp.concatenate(inps, axis=0)
        gi = _contract_last(chunk.astype(jnp.bfloat16), wih_ref[...])
        gi = (gi + bih_ref[...]).reshape(_CB, G, H3)
        gi_ref[pl.ds(i * _CB, _CB)] = gi

    @pl.when(i == na)
    def _init_h():
        h_ref[...] = jnp.zeros_like(h_ref)

    @pl.when(i >= na)
    def _phase_b():
        h = h_ref[...]
        for j in range(_CT):
            t = (i - na) * _CT + j
            gi = gi_ref[:, t, :]
            gh = _contract_last(h.astype(jnp.bfloat16),
                                whh_ref[...]) + bhh_ref[...]
            r = jax.nn.sigmoid(gi[:, :H] + gh[:, :H])
            z = jax.nn.sigmoid(gi[:, H:2 * H] + gh[:, H:2 * H])
            n = jnp.tanh(gi[:, 2 * H:] + r * gh[:, 2 * H:])
            h_new = (1.0 - z) * n + z * h
            mask = t < gc_ref[...]
            out_ref[:, t, :] = jnp.where(mask, h_new, 0.0)
            h = jnp.where(mask, h_new, h)
        h_ref[...] = h


def kernel(enc_states, segments, group_count, W_ih, W_hh, b_ih, b_hh):
    B, S, D = enc_states.shape
    G = segments.shape[1]
    H = W_hh.shape[1]

    wih = W_ih.astype(jnp.bfloat16)
    whh = W_hh.astype(jnp.bfloat16)
    bih = b_ih.reshape(1, 3 * H)
    bhh = b_hh.reshape(1, 3 * H)
    gc = group_count.reshape(B, 1)

    na = B // _CB
    last_a = na - 1
    out = pl.pallas_call(
        _fused_kernel,
        grid=(na + G // _CT,),
        in_specs=[
            pl.BlockSpec((_CB, G, S), lambda i: (jnp.minimum(i, last_a), 0, 0)),
            pl.BlockSpec((_CB, S, D), lambda i: (jnp.minimum(i, last_a), 0, 0)),
            pl.BlockSpec((3 * H, D), lambda i: (0, 0)),
            pl.BlockSpec((3 * H, H), lambda i: (0, 0)),
            pl.BlockSpec((1, 3 * H), lambda i: (0, 0)),
            pl.BlockSpec((1, 3 * H), lambda i: (0, 0)),
            pl.BlockSpec((B, 1), lambda i: (0, 0)),
        ],
        out_specs=pl.BlockSpec((B, G, H), lambda i: (0, 0, 0)),
        out_shape=jax.ShapeDtypeStruct((B, G, H), jnp.float32),
        scratch_shapes=[
            pltpu.VMEM((B, G, 3 * H), jnp.float32),
            pltpu.VMEM((B, H), jnp.float32),
        ],
        compiler_params=pltpu.CompilerParams(
            dimension_semantics=("arbitrary",)),
    )(segments, enc_states, wih, whh, bih, bhh, gc)
    return out


# speedup vs baseline: 1.3299x; 1.0155x over previous
import jax
import jax.numpy as jnp
from jax.experimental import pallas as pl
from jax.experimental.pallas import tpu as pltpu

_CB = 8
_CT = 4


def _contract_last(x, w):
    return jax.lax.dot_general(
        x, w, dimension_numbers=(((1,), (1,)), ((), ())),
        preferred_element_type=jnp.float32)


def _fused_kernel(seg_ref, enc_ref, wih_ref, whh_ref, bih_ref, bhh_ref,
                  gc_ref, out_ref, gi_ref, h_ref):
    B, G, H3 = gi_ref.shape
    H = H3 // 3
    na = B // _CB
    i = pl.program_id(0)

    @pl.when(i < na)
    def _phase_a():
        inps = []
        for j in range(_CB):
            seg = seg_ref[j]
            segb = seg.astype(jnp.bfloat16)
            pooled = jnp.dot(segb, enc_ref[j].astype(jnp.bfloat16),
                             preferred_element_type=jnp.float32)
            denom = jnp.sum(seg, axis=1, keepdims=True).astype(jnp.float32)
            inps.append(pooled * (1.0 / (denom + 1.0)))
        chunk = jnp.concatenate(inps, axis=0)
        gi = _contract_last(chunk.astype(jnp.bfloat16), wih_ref[...])
        gi = (gi + bih_ref[...]).reshape(_CB, G, H3)
        gi_ref[pl.ds(i * _CB, _CB)] = gi

    @pl.when(i == na)
    def _init_h():
        h_ref[...] = jnp.zeros_like(h_ref)

    @pl.when(i >= na)
    def _phase_b():
        h = h_ref[...]
        for j in range(_CT):
            t = (i - na) * _CT + j
            gi = gi_ref[:, t, :]
            gh = _contract_last(h.astype(jnp.bfloat16),
                                whh_ref[...]) + bhh_ref[...]
            r = jax.nn.sigmoid(gi[:, :H] + gh[:, :H])
            z = jax.nn.sigmoid(gi[:, H:2 * H] + gh[:, H:2 * H])
            n = jnp.tanh(gi[:, 2 * H:] + r * gh[:, 2 * H:])
            h_new = (1.0 - z) * n + z * h
            mask = t < gc_ref[...]
            out_ref[:, t, :] = jnp.where(mask, h_new, 0.0)
            h = jnp.where(mask, h_new, h)
        h_ref[...] = h


def kernel(enc_states, segments, group_count, W_ih, W_hh, b_ih, b_hh):
    B, S, D = enc_states.shape
    G = segments.shape[1]
    H = W_hh.shape[1]

    wih = W_ih.astype(jnp.bfloat16)
    whh = W_hh.astype(jnp.bfloat16)
    bih = b_ih.reshape(1, 3 * H)
    bhh = b_hh.reshape(1, 3 * H)
    gc = group_count.reshape(B, 1)

    na = B // _CB
    last_a = na - 1
    out = pl.pallas_call(
        _fused_kernel,
        grid=(na + G // _CT,),
        in_specs=[
            pl.BlockSpec((_CB, G, S), lambda i: (jnp.minimum(i, last_a), 0, 0)),
            pl.BlockSpec((_CB, S, D), lambda i: (jnp.minimum(i, last_a), 0, 0)),
            pl.BlockSpec((3 * H, D), lambda i: (0, 0)),
            pl.BlockSpec((3 * H, H), lambda i: (0, 0)),
            pl.BlockSpec((1, 3 * H), lambda i: (0, 0)),
            pl.BlockSpec((1, 3 * H), lambda i: (0, 0)),
            pl.BlockSpec((B, 1), lambda i: (0, 0)),
        ],
        out_specs=pl.BlockSpec((B, G, H), lambda i: (0, 0, 0)),
        out_shape=jax.ShapeDtypeStruct((B, G, H), jnp.float32),
        scratch_shapes=[
            pltpu.VMEM((B, G, 3 * H), jnp.float32),
            pltpu.VMEM((B, H), jnp.float32),
        ],
        compiler_params=pltpu.CompilerParams(
            dimension_semantics=("arbitrary",)),
    )(segments, enc_states, wih, whh, bih, bhh, gc)
    return out
